# baseline (speedup 1.0000x reference)
import jax
import jax.numpy as jnp
from jax import lax
from jax.experimental import pallas as pl
from jax.experimental.pallas import tpu as pltpu

_ROWS, _RANK, _COLS = 4096, 64, 512
_LTR = 0.01
_ALPHA = 0.1
_TOL = 1e-8
_MAXIT = 20
_K1 = 1.0
_KPOW = 32
_BLK = 512
_NBLK = _ROWS // _BLK
_CH = 512
_NCH = _ROWS // _CH
_DENOM = float(_RANK * _ROWS)
_LSCALE = 256.0


def _setup_kernel(om_ref, z_ref, h_ref, hf8_ref, htf8_ref,
                  mt_ref, vt_ref, lam_ref):
    h = h_ref[...]
    hf8 = hf8_ref[...]
    htf8 = htf8_ref[...]
    m32 = om_ref[...].astype(jnp.float32)
    mT = m32.T
    mt_ref[...] = mT.astype(jnp.bfloat16)

    vu = jnp.dot(m32 * z_ref[...], h.T,
                 preferred_element_type=jnp.float32)
    vt_ref[...] = vu.T

    v0 = jnp.dot(h * h, mT, preferred_element_type=jnp.float32) + _ALPHA

    def apply8(v8):
        t = jnp.dot(htf8, v8, preferred_element_type=jnp.float32)
        tm = (t * mT).astype(jnp.float8_e4m3fn)
        return jnp.dot(hf8, tm, preferred_element_type=jnp.float32) \
            + _ALPHA * v8.astype(jnp.float32)

    def body(_, v8):
        w = apply8(v8)
        return (w / jnp.max(jnp.abs(w), axis=0, keepdims=True)
                ).astype(jnp.float8_e4m3fn)

    v8 = lax.fori_loop(0, _KPOW, body,
                       (v0 / jnp.max(v0, axis=0, keepdims=True)
                        ).astype(jnp.float8_e4m3fn))
    v32 = v8.astype(jnp.float32)
    t = jnp.dot(htf8, v8, preferred_element_type=jnp.float32)
    w = jnp.dot(h, t * mT, preferred_element_type=jnp.float32) + _ALPHA * v32
    lam_ref[...] = (jnp.sum(v32 * w, axis=0, keepdims=True)
                    / jnp.sum(v32 * v32, axis=0, keepdims=True))


def _main_kernel(lt_ref, mt_ref, h_ref, hbf_ref, htbf_ref, vt_ref, lam_ref,
                 wp_ref, out_ref, w_scr, wn_scr):
    w_scr[...] = wp_ref[...]
    inv_lam = 1.0 / lam_ref[...]
    vt = vt_ref[...]
    h = h_ref[...]
    hbf = hbf_ref[...]
    htbf = htbf_ref[...]
    ltr_s = _LTR / _LSCALE

    def step():
        wT = w_scr[...]
        wbf = wT.astype(jnp.bfloat16)
        wf8 = wT.astype(jnp.float8_e4m3fn)
        sse_vec = jnp.zeros((1, _CH), jnp.float32)
        for c in range(_NCH):
            sl = slice(c * _CH, (c + 1) * _CH)
            gt = jnp.dot(wf8, lt_ref[:, sl],
                         preferred_element_type=jnp.float32)
            tt = jnp.dot(htbf, wbf[:, sl],
                         preferred_element_type=jnp.float32)
            tm = tt.astype(jnp.bfloat16) * mt_ref[:, sl]
            ut = jnp.dot(hbf, tm,
                         preferred_element_type=jnp.float32)
            wc = wT[:, sl]
            wn = jax.nn.relu(wc + (vt[:, sl] - ut - _ALPHA * wc - ltr_s * gt)
                             * inv_lam[:, sl])
            wn_scr[:, sl] = wn
            d = wn - wc
            sse_vec = sse_vec + jnp.sum(d * d, axis=0, keepdims=True)
        return jnp.sum(sse_vec)

    def body(_, done):
        sse = step()

        @pl.when(jnp.logical_not(done))
        def _():
            w_scr[...] = wn_scr[...]

        return done | (sse / _DENOM <= _TOL)

    lax.fori_loop(0, _MAXIT - 1, body, jnp.asarray(False))
    step()
    out_ref[...] = wn_scr[...]


def _setup_call(Omega, Z, H, H_f8, Ht_f8):
    return pl.pallas_call(
        _setup_kernel,
        in_specs=[pl.BlockSpec(memory_space=pltpu.VMEM)] * 5,
        out_specs=[pl.BlockSpec(memory_space=pltpu.VMEM)] * 3,
        out_shape=[
            jax.ShapeDtypeStruct((_COLS, _ROWS), jnp.bfloat16),
            jax.ShapeDtypeStruct((_RANK, _ROWS), jnp.float32),
            jax.ShapeDtypeStruct((1, _ROWS), jnp.float32),
        ],
        compiler_params=pltpu.CompilerParams(
            vmem_limit_bytes=56 * 1024 * 1024,
        ),
        name="wlayer_setup",
    )(Omega, Z, H, H_f8, Ht_f8)


def _main_call(LT_f8, mT_bf, H, H_bf, Ht_bf, VT, lamT, WpT):
    return pl.pallas_call(
        _main_kernel,
        in_specs=[
            pl.BlockSpec(memory_space=pltpu.VMEM),
            pl.BlockSpec(memory_space=pltpu.VMEM),
            pl.BlockSpec(memory_space=pltpu.VMEM),
            pl.BlockSpec(memory_space=pltpu.VMEM),
            pl.BlockSpec(memory_space=pltpu.VMEM),
            pl.BlockSpec(memory_space=pltpu.VMEM),
            pl.BlockSpec(memory_space=pltpu.VMEM),
            pl.BlockSpec(memory_space=pltpu.VMEM),
        ],
        out_specs=pl.BlockSpec(memory_space=pltpu.VMEM),
        out_shape=jax.ShapeDtypeStruct((_RANK, _ROWS), jnp.float32),
        scratch_shapes=[
            pltpu.VMEM((_RANK, _ROWS), jnp.float32),
            pltpu.VMEM((_RANK, _ROWS), jnp.float32),
        ],
        compiler_params=pltpu.CompilerParams(
            vmem_limit_bytes=56 * 1024 * 1024,
        ),
        name="wlayer_main",
    )(LT_f8, mT_bf, H, H_bf, Ht_bf, VT, lamT, WpT)


def kernel(Omega, W_pre, H, L, Z):
    LT_f8 = (L.T * _LSCALE).astype(jnp.float8_e4m3fn)
    H_bf = H.astype(jnp.bfloat16)
    Ht_bf = H_bf.T
    H_f8 = H.astype(jnp.float8_e4m3fn)
    Ht_f8 = H_f8.T
    WpT = W_pre.T

    mT_bf, VT, lamT = _setup_call(Omega, Z, H, H_f8, Ht_f8)
    outT = _main_call(LT_f8, mT_bf, H, H_bf, Ht_bf, VT, lamT, WpT)
    return _K1 * outT.T


# speedup vs baseline: 1.2102x; 1.2102x over previous
import jax
import jax.numpy as jnp
from jax import lax
from jax.experimental import pallas as pl
from jax.experimental.pallas import tpu as pltpu

_ROWS, _RANK, _COLS = 4096, 64, 512
_LTR = 0.01
_ALPHA = 0.1
_TOL = 1e-8
_MAXIT = 20
_K1 = 1.0
_KPOW = 32
_BLK = 512
_NBLK = _ROWS // _BLK
_CH = 1024
_NCH = _ROWS // _CH
_DENOM = float(_RANK * _ROWS)
_LSCALE = 256.0


def _setup_kernel(om_ref, z_ref, h_ref, hf8_ref, htf8_ref,
                  mt_ref, vt_ref, lam_ref):
    h = h_ref[...]
    hf8 = hf8_ref[...]
    htf8 = htf8_ref[...]
    m32 = om_ref[...].astype(jnp.float32)
    mT = m32.T
    mt_ref[...] = mT.astype(jnp.bfloat16)

    vu = jnp.dot(m32 * z_ref[...], h.T,
                 preferred_element_type=jnp.float32)
    vt_ref[...] = vu.T

    v0 = jnp.dot(h * h, mT, preferred_element_type=jnp.float32) + _ALPHA

    def apply8(v8):
        t = jnp.dot(htf8, v8, preferred_element_type=jnp.float32)
        tm = (t * mT).astype(jnp.float8_e4m3fn)
        return jnp.dot(hf8, tm, preferred_element_type=jnp.float32) \
            + _ALPHA * v8.astype(jnp.float32)

    def body(_, v8):
        w = apply8(v8)
        return (w / jnp.max(jnp.abs(w), axis=0, keepdims=True)
                ).astype(jnp.float8_e4m3fn)

    v8 = lax.fori_loop(0, _KPOW, body,
                       (v0 / jnp.max(v0, axis=0, keepdims=True)
                        ).astype(jnp.float8_e4m3fn))
    v32 = v8.astype(jnp.float32)
    t = jnp.dot(htf8, v8, preferred_element_type=jnp.float32)
    w = jnp.dot(h, t * mT, preferred_element_type=jnp.float32) + _ALPHA * v32
    lam_ref[...] = (jnp.sum(v32 * w, axis=0, keepdims=True)
                    / jnp.sum(v32 * v32, axis=0, keepdims=True))


def _main_kernel(lt_ref, mt_ref, h_ref, hbf_ref, htbf_ref, vt_ref, lam_ref,
                 wp_ref, out_ref, w_scr, wn_scr):
    w_scr[...] = wp_ref[...]
    inv_lam = 1.0 / lam_ref[...]
    vt = vt_ref[...]
    h = h_ref[...]
    hbf = hbf_ref[...]
    htbf = htbf_ref[...]
    ltr_s = _LTR / _LSCALE

    def step():
        wT = w_scr[...]
        wbf = wT.astype(jnp.bfloat16)
        wf8 = wT.astype(jnp.float8_e4m3fn)
        sse_vec = jnp.zeros((1, _CH), jnp.float32)
        for c in range(_NCH):
            sl = slice(c * _CH, (c + 1) * _CH)
            gt = jnp.dot(wf8, lt_ref[:, sl],
                         preferred_element_type=jnp.float32)
            tt = jnp.dot(htbf, wbf[:, sl],
                         preferred_element_type=jnp.float32)
            tm = tt.astype(jnp.bfloat16) * mt_ref[:, sl]
            ut = jnp.dot(hbf, tm,
                         preferred_element_type=jnp.float32)
            wc = wT[:, sl]
            wn = jax.nn.relu(wc + (vt[:, sl] - ut - _ALPHA * wc - ltr_s * gt)
                             * inv_lam[:, sl])
            wn_scr[:, sl] = wn
            d = wn - wc
            sse_vec = sse_vec + jnp.sum(d * d, axis=0, keepdims=True)
        return jnp.sum(sse_vec)

    def body(_, done):
        sse = step()

        @pl.when(jnp.logical_not(done))
        def _():
            w_scr[...] = wn_scr[...]

        return done | (sse / _DENOM <= _TOL)

    lax.fori_loop(0, _MAXIT - 1, body, jnp.asarray(False))
    step()
    out_ref[...] = wn_scr[...]


def _lprep_kernel(l_ref, lt_ref):
    for j in range(_ROWS // _BLK):
        t = l_ref[pl.ds(j * _BLK, _BLK), :].T
        lt_ref[:, pl.ds(j * _BLK, _BLK)] = (
            (t * _LSCALE).astype(jnp.float8_e4m3fn))


def _lprep_call(L):
    return pl.pallas_call(
        _lprep_kernel,
        grid=(_NBLK,),
        in_specs=[pl.BlockSpec((_ROWS, _BLK), lambda i: (0, i))],
        out_specs=pl.BlockSpec((_BLK, _ROWS), lambda i: (i, 0)),
        out_shape=jax.ShapeDtypeStruct((_ROWS, _ROWS), jnp.float8_e4m3fn),
        compiler_params=pltpu.CompilerParams(
            dimension_semantics=("arbitrary",),
        ),
        name="wlayer_lprep",
    )(L)


def _setup_call(Omega, Z, H, H_f8, Ht_f8):
    return pl.pallas_call(
        _setup_kernel,
        in_specs=[pl.BlockSpec(memory_space=pltpu.VMEM)] * 5,
        out_specs=[pl.BlockSpec(memory_space=pltpu.VMEM)] * 3,
        out_shape=[
            jax.ShapeDtypeStruct((_COLS, _ROWS), jnp.bfloat16),
            jax.ShapeDtypeStruct((_RANK, _ROWS), jnp.float32),
            jax.ShapeDtypeStruct((1, _ROWS), jnp.float32),
        ],
        compiler_params=pltpu.CompilerParams(
            vmem_limit_bytes=56 * 1024 * 1024,
        ),
        name="wlayer_setup",
    )(Omega, Z, H, H_f8, Ht_f8)


def _main_call(LT_f8, mT_bf, H, H_bf, Ht_bf, VT, lamT, WpT):
    return pl.pallas_call(
        _main_kernel,
        in_specs=[
            pl.BlockSpec(memory_space=pltpu.VMEM),
            pl.BlockSpec(memory_space=pltpu.VMEM),
            pl.BlockSpec(memory_space=pltpu.VMEM),
            pl.BlockSpec(memory_space=pltpu.VMEM),
            pl.BlockSpec(memory_space=pltpu.VMEM),
            pl.BlockSpec(memory_space=pltpu.VMEM),
            pl.BlockSpec(memory_space=pltpu.VMEM),
            pl.BlockSpec(memory_space=pltpu.VMEM),
        ],
        out_specs=pl.BlockSpec(memory_space=pltpu.VMEM),
        out_shape=jax.ShapeDtypeStruct((_RANK, _ROWS), jnp.float32),
        scratch_shapes=[
            pltpu.VMEM((_RANK, _ROWS), jnp.float32),
            pltpu.VMEM((_RANK, _ROWS), jnp.float32),
        ],
        compiler_params=pltpu.CompilerParams(
            vmem_limit_bytes=56 * 1024 * 1024,
        ),
        name="wlayer_main",
    )(LT_f8, mT_bf, H, H_bf, Ht_bf, VT, lamT, WpT)


def kernel(Omega, W_pre, H, L, Z):
    LT_f8 = _lprep_call(L)
    H_bf = H.astype(jnp.bfloat16)
    Ht_bf = H_bf.T
    H_f8 = H.astype(jnp.float8_e4m3fn)
    Ht_f8 = H_f8.T
    WpT = W_pre.T

    mT_bf, VT, lamT = _setup_call(Omega, Z, H, H_f8, Ht_f8)
    outT = _main_call(LT_f8, mT_bf, H, H_bf, Ht_bf, VT, lamT, WpT)
    return _K1 * outT.T


# speedup vs baseline: 1.2144x; 1.0035x over previous
import jax
import jax.numpy as jnp
from jax import lax
from jax.experimental import pallas as pl
from jax.experimental.pallas import tpu as pltpu

_ROWS, _RANK, _COLS = 4096, 64, 512
_LTR = 0.01
_ALPHA = 0.1
_TOL = 1e-8
_MAXIT = 20
_K1 = 1.0
_KPOW = 32
_BLK = 512
_NBLK = _ROWS // _BLK
_CH = 1024
_NCH = _ROWS // _CH
_DENOM = float(_RANK * _ROWS)
_LSCALE = 256.0


def _setup_kernel(om_ref, z_ref, h_ref, hf8_ref, htf8_ref,
                  mt_ref, vt_ref, lam_ref):
    h = h_ref[...]
    hf8 = hf8_ref[...]
    htf8 = htf8_ref[...]
    m32 = om_ref[...].astype(jnp.float32)
    mT = m32.T
    mt_ref[...] = mT.astype(jnp.bfloat16)

    vu = jnp.dot(m32 * z_ref[...], h.T,
                 preferred_element_type=jnp.float32)
    vt_ref[...] = vu.T

    v0 = jnp.dot(h * h, mT, preferred_element_type=jnp.float32) + _ALPHA

    def apply8(v8):
        t = jnp.dot(htf8, v8, preferred_element_type=jnp.float32)
        tm = (t * mT).astype(jnp.float8_e4m3fn)
        return jnp.dot(hf8, tm, preferred_element_type=jnp.float32) \
            + _ALPHA * v8.astype(jnp.float32)

    def body(_, v8):
        w = apply8(v8)
        return (w / jnp.max(jnp.abs(w), axis=0, keepdims=True)
                ).astype(jnp.float8_e4m3fn)

    v8 = lax.fori_loop(0, _KPOW, body,
                       (v0 / jnp.max(v0, axis=0, keepdims=True)
                        ).astype(jnp.float8_e4m3fn))
    v32 = v8.astype(jnp.float32)
    t = jnp.dot(htf8, v8, preferred_element_type=jnp.float32)
    w = jnp.dot(h, t * mT, preferred_element_type=jnp.float32) + _ALPHA * v32
    lam_ref[...] = (jnp.sum(v32 * w, axis=0, keepdims=True)
                    / jnp.sum(v32 * v32, axis=0, keepdims=True))


def _main_kernel(lt_ref, mt_ref, h_ref, hbf_ref, htbf_ref, vt_ref, lam_ref,
                 wp_ref, out_ref, w_scr, wn_scr):
    w_scr[...] = wp_ref[...]
    inv_lam = 1.0 / lam_ref[...]
    vt = vt_ref[...]
    h = h_ref[...]
    hbf = hbf_ref[...]
    htbf = htbf_ref[...]
    ltr_s = _LTR / _LSCALE

    def step():
        wT = w_scr[...]
        wbf = wT.astype(jnp.bfloat16)
        wf8 = wT.astype(jnp.float8_e4m3fn)
        sse_vec = jnp.zeros((1, _CH), jnp.float32)
        for c in range(_NCH):
            sl = slice(c * _CH, (c + 1) * _CH)
            gt = jnp.dot(wf8, lt_ref[:, sl],
                         preferred_element_type=jnp.float32)
            tt = jnp.dot(htbf, wbf[:, sl],
                         preferred_element_type=jnp.float32)
            tm = tt.astype(jnp.bfloat16) * mt_ref[:, sl]
            ut = jnp.dot(hbf, tm,
                         preferred_element_type=jnp.float32)
            wc = wT[:, sl]
            wn = jax.nn.relu(wc + (vt[:, sl] - ut - _ALPHA * wc - ltr_s * gt)
                             * inv_lam[:, sl])
            wn_scr[:, sl] = wn
            d = wn - wc
            sse_vec = sse_vec + jnp.sum(d * d, axis=0, keepdims=True)
        return jnp.sum(sse_vec)

    def body(_, done):
        sse = step()

        @pl.when(jnp.logical_not(done))
        def _():
            w_scr[...] = wn_scr[...]

        return done | (sse / _DENOM <= _TOL)

    lax.fori_loop(0, _MAXIT - 1, body, jnp.asarray(False))
    step()
    out_ref[...] = wn_scr[...]


def _lprep_kernel(l_ref, lt_ref):
    for j in range(_ROWS // _BLK):
        t = l_ref[:, pl.ds(j * _BLK, _BLK)].T
        lt_ref[pl.ds(j * _BLK, _BLK), :] = (
            (t * _LSCALE).astype(jnp.float8_e4m3fn))


def _lprep_call(L):
    return pl.pallas_call(
        _lprep_kernel,
        grid=(_NBLK,),
        in_specs=[pl.BlockSpec((_BLK, _ROWS), lambda i: (i, 0))],
        out_specs=pl.BlockSpec((_ROWS, _BLK), lambda i: (0, i)),
        out_shape=jax.ShapeDtypeStruct((_ROWS, _ROWS), jnp.float8_e4m3fn),
        compiler_params=pltpu.CompilerParams(
            dimension_semantics=("arbitrary",),
        ),
        name="wlayer_lprep",
    )(L)


def _setup_call(Omega, Z, H, H_f8, Ht_f8):
    return pl.pallas_call(
        _setup_kernel,
        in_specs=[pl.BlockSpec(memory_space=pltpu.VMEM)] * 5,
        out_specs=[pl.BlockSpec(memory_space=pltpu.VMEM)] * 3,
        out_shape=[
            jax.ShapeDtypeStruct((_COLS, _ROWS), jnp.bfloat16),
            jax.ShapeDtypeStruct((_RANK, _ROWS), jnp.float32),
            jax.ShapeDtypeStruct((1, _ROWS), jnp.float32),
        ],
        compiler_params=pltpu.CompilerParams(
            vmem_limit_bytes=56 * 1024 * 1024,
        ),
        name="wlayer_setup",
    )(Omega, Z, H, H_f8, Ht_f8)


def _main_call(LT_f8, mT_bf, H, H_bf, Ht_bf, VT, lamT, WpT):
    return pl.pallas_call(
        _main_kernel,
        in_specs=[
            pl.BlockSpec(memory_space=pltpu.VMEM),
            pl.BlockSpec(memory_space=pltpu.VMEM),
            pl.BlockSpec(memory_space=pltpu.VMEM),
            pl.BlockSpec(memory_space=pltpu.VMEM),
            pl.BlockSpec(memory_space=pltpu.VMEM),
            pl.BlockSpec(memory_space=pltpu.VMEM),
            pl.BlockSpec(memory_space=pltpu.VMEM),
            pl.BlockSpec(memory_space=pltpu.VMEM),
        ],
        out_specs=pl.BlockSpec(memory_space=pltpu.VMEM),
        out_shape=jax.ShapeDtypeStruct((_RANK, _ROWS), jnp.float32),
        scratch_shapes=[
            pltpu.VMEM((_RANK, _ROWS), jnp.float32),
            pltpu.VMEM((_RANK, _ROWS), jnp.float32),
        ],
        compiler_params=pltpu.CompilerParams(
            vmem_limit_bytes=56 * 1024 * 1024,
        ),
        name="wlayer_main",
    )(LT_f8, mT_bf, H, H_bf, Ht_bf, VT, lamT, WpT)


def kernel(Omega, W_pre, H, L, Z):
    LT_f8 = _lprep_call(L)
    H_bf = H.astype(jnp.bfloat16)
    Ht_bf = H_bf.T
    H_f8 = H.astype(jnp.float8_e4m3fn)
    Ht_f8 = H_f8.T
    WpT = W_pre.T

    mT_bf, VT, lamT = _setup_call(Omega, Z, H, H_f8, Ht_f8)
    outT = _main_call(LT_f8, mT_bf, H, H_bf, Ht_bf, VT, lamT, WpT)
    return _K1 * outT.T
